# baseline (speedup 1.0000x reference)
import functools

import jax
import jax.numpy as jnp
from jax import lax
from jax.experimental import pallas as pl
from jax.experimental.pallas import tpu as pltpu
from jax.experimental.pallas import tpu_sc as plsc

N = 10000
NPAD = 10112
E = 320000
F = 128
FC = 144
NCLASS = 40
NS = 16
C = 128
NBUF = 2
G = 8
NG = 20
CH = NG * G
EPT = CH * C
RT = NPAD // NS
FLUSH = tuple((o, min(64, RT - o)) for o in range(0, RT, 64))
BR = 1000
NB = N // BR
P = 2.5


def _zero_rows(ref, rows, width):
    def outer(i, _):
        for j in range(width // 16):
            ref[i, pl.ds(j * 16, 16)] = jnp.zeros((16,), jnp.float32)
        return 0
    lax.fori_loop(0, rows, outer, 0)


def _mesh():
    return plsc.VectorSubcoreMesh(core_axis_name="c", subcore_axis_name="s")


def _sc_degrees(srcs3, dsts3):
    @functools.partial(
        pl.kernel,
        out_type=(jax.ShapeDtypeStruct((NPAD, 16), jnp.float32),
                  jax.ShapeDtypeStruct((NPAD, 16), jnp.float32)),
        mesh=_mesh(),
        scratch_types=[
            pltpu.VMEM((CH, C), jnp.int32),
            pltpu.VMEM((C, 16), jnp.float32),
            pltpu.VMEM((128, 16), jnp.float32),
            pltpu.VMEM_SHARED((NPAD, 16), jnp.float32),
        ],
        compiler_params=pltpu.CompilerParams(use_tc_tiling_on_sc=False),
    )
    def deg_kernel(src_hbm, dst_hbm, degin_hbm, degout_hbm,
                   idx_v, ones_v, stage_v, acc_sh):
        c = lax.axis_index("c")
        s = lax.axis_index("s")

        def fill_ones(i, _):
            ones_v[i, :] = jnp.ones((16,), jnp.float32)
            return 0
        lax.fori_loop(0, C, fill_ones, 0)
        _zero_rows(stage_v, 128, 16)

        base = s * RT
        for off, sz in FLUSH:
            pltpu.sync_copy(stage_v.at[:sz], acc_sh.at[pl.ds(base + off, sz)])
        plsc.subcore_barrier()

        def run(idx_hbm, out_hbm):
            pltpu.sync_copy(idx_hbm.at[s], idx_v)

            def chunk(j, _):
                pltpu.sync_copy(ones_v, acc_sh.at[idx_v.at[j]], add=True)
                return 0
            lax.fori_loop(0, CH, chunk, 0)
            plsc.subcore_barrier()
            for off, sz in FLUSH:
                r0 = base + off
                pltpu.sync_copy(acc_sh.at[pl.ds(r0, sz)], stage_v.at[:sz])
                pltpu.sync_copy(stage_v.at[:sz], out_hbm.at[pl.ds(r0, sz)])

        @pl.when(c == 0)
        def _():
            run(dst_hbm, degin_hbm)

        @pl.when(c == 1)
        def _():
            run(src_hbm, degout_hbm)

    return deg_kernel(srcs3, dsts3)


def _sc_aggregate(table0, table1, srcg3, srcs3, dstg3, dsts3):
    @functools.partial(
        pl.kernel,
        out_type=(jax.ShapeDtypeStruct((NPAD, FC), jnp.float32),
                  jax.ShapeDtypeStruct((NPAD, FC), jnp.float32)),
        mesh=_mesh(),
        scratch_types=[
            pltpu.VMEM((G, C), jnp.int32),
            pltpu.VMEM((G, C), jnp.int32),
            pltpu.VMEM((C, FC), jnp.float32),
            pltpu.VMEM((C, FC), jnp.float32),
            pltpu.VMEM_SHARED((NPAD, FC), jnp.float32),
            pltpu.SemaphoreType.DMA((NBUF,)),
            pltpu.SemaphoreType.DMA((NBUF,)),
        ],
        compiler_params=pltpu.CompilerParams(use_tc_tiling_on_sc=False),
    )
    def agg_kernel(t0_hbm, t1_hbm, srcg_hbm, srcs_hbm, dstg_hbm, dsts_hbm,
                   out0_hbm, out1_hbm,
                   idxg_v, idxs_v, buf0_v, buf1_v, acc_sh,
                   gsem, ssem):
        c = lax.axis_index("c")
        s = lax.axis_index("s")
        bufs = (buf0_v, buf1_v)

        _zero_rows(buf0_v, C, FC)
        base = s * RT
        for off, sz in FLUSH:
            pltpu.sync_copy(buf0_v.at[:sz], acc_sh.at[pl.ds(base + off, sz)])
        plsc.subcore_barrier()

        def run(table_hbm, ig_hbm, is_hbm, out_hbm):
            def group(g, _):
                pltpu.sync_copy(ig_hbm.at[s, pl.ds(g * G, G)], idxg_v)
                pltpu.sync_copy(is_hbm.at[s, pl.ds(g * G, G)], idxs_v)
                gd = [None] * G
                sd = [None] * G
                for k in range(NBUF - 1):
                    gd[k] = pltpu.async_copy(
                        table_hbm.at[idxg_v.at[k]], bufs[k], gsem.at[k])
                for j in range(G):
                    b = j % NBUF
                    gd[j].wait()
                    sd[j] = pltpu.async_copy(
                        bufs[b], acc_sh.at[idxs_v.at[j]], ssem.at[b],
                        add=True)
                    jn = j + NBUF - 1
                    if jn < G:
                        if j >= 1:
                            sd[j - 1].wait()
                        gd[jn] = pltpu.async_copy(
                            table_hbm.at[idxg_v.at[jn]], bufs[jn % NBUF],
                            gsem.at[jn % NBUF])
                for j in range(G - NBUF, G):
                    if j >= 0:
                        sd[j].wait()
                return 0
            lax.fori_loop(0, NG, group, 0)
            plsc.subcore_barrier()
            for off, sz in FLUSH:
                r0 = base + off
                pltpu.sync_copy(acc_sh.at[pl.ds(r0, sz)], buf0_v.at[:sz])
                pltpu.sync_copy(buf0_v.at[:sz], out_hbm.at[pl.ds(r0, sz)])

        @pl.when(c == 0)
        def _():
            run(t0_hbm, srcg_hbm, dsts_hbm, out0_hbm)

        @pl.when(c == 1)
        def _():
            run(t1_hbm, dstg_hbm, srcs_hbm, out1_hbm)

    return agg_kernel(table0, table1, srcg3, srcs3, dstg3, dsts3)


_DN = (((1,), (1,)), ((), ()))


def _emit_layer_pre(hb, w_ref, di_ref, hw_ref, hw2_ref, hcat_ref, cs_ref, i):
    hw = lax.dot_general(hb, w_ref[...], _DN,
                         preferred_element_type=jnp.float32)
    hw_ref[...] = hw
    dinv = lax.rsqrt(di_ref[...][:, 0:1] + 1.0)
    hw2_ref[...] = jnp.concatenate(
        [hw * dinv, jnp.zeros((BR, FC - F), jnp.float32)], axis=1)
    q = jnp.sum(hb * hb, axis=1, keepdims=True)
    hcat_ref[...] = jnp.concatenate(
        [hb, q, jnp.zeros((BR, FC - F - 1), jnp.float32)], axis=1)

    @pl.when(i == 0)
    def _():
        cs_ref[...] = jnp.sum(hb, axis=0, keepdims=True)

    @pl.when(i != 0)
    def _():
        cs_ref[...] += jnp.sum(hb, axis=0, keepdims=True)


def _combine(h_ref, hw_ref, a1_ref, a2_ref, xs_ref, di_ref, do_ref,
             cb_ref, cs_ref):
    hb = h_ref[...]
    gm = cs_ref[...] * (1.0 / N)
    dinv = lax.rsqrt(di_ref[...][:, 0:1] + 1.0)
    dout = do_ref[...][:, 0:1]
    a1 = a1_ref[...][:, :F]
    x_agg = jnp.maximum(
        dinv * a1 + (dinv * dinv) * hw_ref[...] + cb_ref[...], 0.0)
    a2full = a2_ref[...]
    agg2 = a2full[:, :F]
    s1 = a2full[:, F:F + 1]
    q = jnp.sum(hb * hb, axis=1, keepdims=True)
    dotv = jnp.sum(hb * agg2, axis=1, keepdims=True)
    gnum = dout * q + s1 - 2.0 * dotv
    gs = jnp.tanh(gnum / (dout + 1e-10))
    d = jnp.sum(jnp.abs(hb - gm) ** P, axis=1, keepdims=True)
    gq = 1.0 - jnp.tanh(d)
    return (hb + gs * x_agg + gq * xs_ref[...]) / (1.0 + gs + gq)


_ROWB = pl.BlockSpec((BR, F), lambda i: (i, 0))
_ROWC = pl.BlockSpec((BR, FC), lambda i: (i, 0))
_ROW16 = pl.BlockSpec((BR, 16), lambda i: (i, 0))
_WB = pl.BlockSpec((F, F), lambda i: (0, 0))
_B1 = pl.BlockSpec((1, F), lambda i: (0, 0))


def _tc_pre(x, enc_w, enc_b2, skip_w, conv_w, deg_in):
    def body(x_ref, ew_ref, eb_ref, sw_ref, w_ref, di_ref,
             h0_ref, xs_ref, hw_ref, hw2_ref, hcat_ref, cs_ref):
        i = pl.program_id(0)
        xb = x_ref[...]
        h0 = jnp.maximum(
            lax.dot_general(xb, ew_ref[...], _DN,
                            preferred_element_type=jnp.float32) + eb_ref[...],
            0.0)
        h0_ref[...] = h0
        xs_ref[...] = lax.dot_general(xb, sw_ref[...], _DN,
                                      preferred_element_type=jnp.float32)
        _emit_layer_pre(h0, w_ref, di_ref, hw_ref, hw2_ref, hcat_ref,
                        cs_ref, i)

    return pl.pallas_call(
        body,
        grid=(NB,),
        in_specs=[_ROWB, _WB, _B1, _WB, _WB, _ROW16],
        out_specs=[_ROWB, _ROWB, _ROWB, _ROWC, _ROWC, _B1],
        out_shape=[jax.ShapeDtypeStruct((N, F), jnp.float32),
                   jax.ShapeDtypeStruct((N, F), jnp.float32),
                   jax.ShapeDtypeStruct((N, F), jnp.float32),
                   jax.ShapeDtypeStruct((N, FC), jnp.float32),
                   jax.ShapeDtypeStruct((N, FC), jnp.float32),
                   jax.ShapeDtypeStruct((1, F), jnp.float32)],
    )(x, enc_w, enc_b2, skip_w, conv_w, deg_in)


def _tc_mid(h, hw, agg1, agg2cat, x_skip, deg_in, deg_out, cb, cs, conv_w):
    def body(h_ref, hw_ref, a1_ref, a2_ref, xs_ref, di_ref, do_ref,
             cb_ref, cs_ref, w_ref,
             h1_ref, hw1_ref, hw2_ref, hcat_ref, cs1_ref):
        i = pl.program_id(0)
        h1 = _combine(h_ref, hw_ref, a1_ref, a2_ref, xs_ref, di_ref, do_ref,
                      cb_ref, cs_ref)
        h1_ref[...] = h1
        _emit_layer_pre(h1, w_ref, di_ref, hw1_ref, hw2_ref, hcat_ref,
                        cs1_ref, i)

    return pl.pallas_call(
        body,
        grid=(NB,),
        in_specs=[_ROWB, _ROWB, _ROWC, _ROWC, _ROWB, _ROW16, _ROW16,
                  _B1, _B1, _WB],
        out_specs=[_ROWB, _ROWB, _ROWC, _ROWC, _B1],
        out_shape=[jax.ShapeDtypeStruct((N, F), jnp.float32),
                   jax.ShapeDtypeStruct((N, F), jnp.float32),
                   jax.ShapeDtypeStruct((N, FC), jnp.float32),
                   jax.ShapeDtypeStruct((N, FC), jnp.float32),
                   jax.ShapeDtypeStruct((1, F), jnp.float32)],
    )(h, hw, agg1, agg2cat, x_skip, deg_in, deg_out, cb, cs, conv_w)


def _tc_final(h, hw, agg1, agg2cat, x_skip, deg_in, deg_out, cb, cs,
              dec_w, dec_b2):
    def body(h_ref, hw_ref, a1_ref, a2_ref, xs_ref, di_ref, do_ref,
             cb_ref, cs_ref, dw_ref, db_ref, out_ref):
        h2 = _combine(h_ref, hw_ref, a1_ref, a2_ref, xs_ref, di_ref, do_ref,
                      cb_ref, cs_ref)
        out_ref[...] = lax.dot_general(
            h2, dw_ref[...], _DN,
            preferred_element_type=jnp.float32) + db_ref[...]

    return pl.pallas_call(
        body,
        grid=(NB,),
        in_specs=[_ROWB, _ROWB, _ROWC, _ROWC, _ROWB, _ROW16, _ROW16,
                  _B1, _B1,
                  pl.BlockSpec((NCLASS, F), lambda i: (0, 0)),
                  pl.BlockSpec((1, NCLASS), lambda i: (0, 0))],
        out_specs=pl.BlockSpec((BR, NCLASS), lambda i: (i, 0)),
        out_shape=jax.ShapeDtypeStruct((N, NCLASS), jnp.float32),
    )(h, hw, agg1, agg2cat, x_skip, deg_in, deg_out, cb, cs, dec_w, dec_b2)


def kernel(x, edge_index, enc_w, enc_b, skip_w, conv_w, conv_b, dec_w, dec_b):
    ept = E // NS
    src2 = edge_index[0].reshape(NS, ept)
    dst2 = edge_index[1].reshape(NS, ept)
    k = jnp.arange(NS * (EPT - ept), dtype=jnp.int32).reshape(NS, EPT - ept)
    padg = k % N
    pads = N + k % (NPAD - N)
    srcg3 = jnp.concatenate([src2, padg], 1).reshape(NS, CH, C)
    srcs3 = jnp.concatenate([src2, pads], 1).reshape(NS, CH, C)
    dstg3 = jnp.concatenate([dst2, padg], 1).reshape(NS, CH, C)
    dsts3 = jnp.concatenate([dst2, pads], 1).reshape(NS, CH, C)
    deg_in, deg_out = _sc_degrees(srcs3, dsts3)
    cb = conv_b.reshape(1, F)
    h, x_skip, hw, hw2, hcat, cs = _tc_pre(
        x, enc_w, enc_b.reshape(1, F), skip_w, conv_w, deg_in)
    agg1, agg2cat = _sc_aggregate(hw2, hcat, srcg3, srcs3, dstg3, dsts3)
    h, hw, hw2, hcat, cs = _tc_mid(
        h, hw, agg1, agg2cat, x_skip, deg_in, deg_out, cb, cs, conv_w)
    agg1, agg2cat = _sc_aggregate(hw2, hcat, srcg3, srcs3, dstg3, dsts3)
    return _tc_final(h, hw, agg1, agg2cat, x_skip, deg_in, deg_out, cb, cs,
                     dec_w, dec_b.reshape(1, NCLASS))


# speedup vs baseline: 1.1058x; 1.1058x over previous
import functools

import jax
import jax.numpy as jnp
from jax import lax
from jax.experimental import pallas as pl
from jax.experimental.pallas import tpu as pltpu
from jax.experimental.pallas import tpu_sc as plsc

N = 10000
NPAD = 10112
E = 320000
F = 128
FC = 144
NCLASS = 40
NS = 16
C = 32
NBUF = 8
G = 32
NG = 20
CH = NG * G
EPT = CH * C
RT = NPAD // NS
FLUSH = tuple((o, min(C, RT - o)) for o in range(0, RT, C))
BR = 1000
NB = N // BR
P = 2.5


def _zero_rows(ref, rows, width):
    def outer(i, _):
        for j in range(width // 16):
            ref[i, pl.ds(j * 16, 16)] = jnp.zeros((16,), jnp.float32)
        return 0
    lax.fori_loop(0, rows, outer, 0)


def _mesh():
    return plsc.VectorSubcoreMesh(core_axis_name="c", subcore_axis_name="s")


def _sc_degrees(srcs3, dsts3):
    @functools.partial(
        pl.kernel,
        out_type=(jax.ShapeDtypeStruct((NPAD, 16), jnp.float32),
                  jax.ShapeDtypeStruct((NPAD, 16), jnp.float32)),
        mesh=_mesh(),
        scratch_types=[
            pltpu.VMEM((CH, C), jnp.int32),
            pltpu.VMEM((C, 16), jnp.float32),
            pltpu.VMEM((128, 16), jnp.float32),
            pltpu.VMEM_SHARED((NPAD, 16), jnp.float32),
        ],
        compiler_params=pltpu.CompilerParams(use_tc_tiling_on_sc=False),
    )
    def deg_kernel(src_hbm, dst_hbm, degin_hbm, degout_hbm,
                   idx_v, ones_v, stage_v, acc_sh):
        c = lax.axis_index("c")
        s = lax.axis_index("s")

        def fill_ones(i, _):
            ones_v[i, :] = jnp.ones((16,), jnp.float32)
            return 0
        lax.fori_loop(0, C, fill_ones, 0)
        _zero_rows(stage_v, 128, 16)

        base = s * RT
        for off, sz in FLUSH:
            pltpu.sync_copy(stage_v.at[:sz], acc_sh.at[pl.ds(base + off, sz)])
        plsc.subcore_barrier()

        def run(idx_hbm, out_hbm):
            pltpu.sync_copy(idx_hbm.at[s], idx_v)

            def chunk(j, _):
                pltpu.sync_copy(ones_v, acc_sh.at[idx_v.at[j]], add=True)
                return 0
            lax.fori_loop(0, CH, chunk, 0)
            plsc.subcore_barrier()
            for off, sz in FLUSH:
                r0 = base + off
                pltpu.sync_copy(acc_sh.at[pl.ds(r0, sz)], stage_v.at[:sz])
                pltpu.sync_copy(stage_v.at[:sz], out_hbm.at[pl.ds(r0, sz)])

        @pl.when(c == 0)
        def _():
            run(dst_hbm, degin_hbm)

        @pl.when(c == 1)
        def _():
            run(src_hbm, degout_hbm)

    return deg_kernel(srcs3, dsts3)


def _sc_aggregate(table0, table1, srcg3, srcs3, dstg3, dsts3):
    @functools.partial(
        pl.kernel,
        out_type=(jax.ShapeDtypeStruct((NPAD, FC), jnp.float32),
                  jax.ShapeDtypeStruct((NPAD, FC), jnp.float32)),
        mesh=_mesh(),
        scratch_types=[
            pltpu.VMEM((G, C), jnp.int32),
            pltpu.VMEM((G, C), jnp.int32),
            pltpu.VMEM((NBUF, C, FC), jnp.float32),
            pltpu.VMEM_SHARED((NPAD, FC), jnp.float32),
            pltpu.SemaphoreType.DMA((NBUF,)),
            pltpu.SemaphoreType.DMA((NBUF,)),
        ],
        compiler_params=pltpu.CompilerParams(use_tc_tiling_on_sc=False),
    )
    def agg_kernel(t0_hbm, t1_hbm, srcg_hbm, srcs_hbm, dstg_hbm, dsts_hbm,
                   out0_hbm, out1_hbm,
                   idxg_v, idxs_v, bufn_v, acc_sh,
                   gsem, ssem):
        c = lax.axis_index("c")
        s = lax.axis_index("s")
        bufs = tuple(bufn_v.at[k] for k in range(NBUF))
        buf0_v = bufn_v.at[0]

        _zero_rows(buf0_v, C, FC)
        base = s * RT
        for off, sz in FLUSH:
            pltpu.sync_copy(buf0_v.at[:sz], acc_sh.at[pl.ds(base + off, sz)])
        plsc.subcore_barrier()

        def run(table_hbm, ig_hbm, is_hbm, out_hbm):
            def group(g, _):
                pltpu.sync_copy(ig_hbm.at[s, pl.ds(g * G, G)], idxg_v)
                pltpu.sync_copy(is_hbm.at[s, pl.ds(g * G, G)], idxs_v)
                gd = [None] * G
                sd = [None] * G
                for k in range(NBUF - 1):
                    gd[k] = pltpu.async_copy(
                        table_hbm.at[idxg_v.at[k]], bufs[k], gsem.at[k])
                for j in range(G):
                    b = j % NBUF
                    gd[j].wait()
                    sd[j] = pltpu.async_copy(
                        bufs[b], acc_sh.at[idxs_v.at[j]], ssem.at[b],
                        add=True)
                    jn = j + NBUF - 1
                    if jn < G:
                        if j >= 1:
                            sd[j - 1].wait()
                        gd[jn] = pltpu.async_copy(
                            table_hbm.at[idxg_v.at[jn]], bufs[jn % NBUF],
                            gsem.at[jn % NBUF])
                for j in range(G - NBUF, G):
                    if j >= 0:
                        sd[j].wait()
                return 0
            lax.fori_loop(0, NG, group, 0)
            plsc.subcore_barrier()
            for off, sz in FLUSH:
                r0 = base + off
                pltpu.sync_copy(acc_sh.at[pl.ds(r0, sz)], buf0_v.at[:sz])
                pltpu.sync_copy(buf0_v.at[:sz], out_hbm.at[pl.ds(r0, sz)])

        @pl.when(c == 0)
        def _():
            run(t0_hbm, srcg_hbm, dsts_hbm, out0_hbm)

        @pl.when(c == 1)
        def _():
            run(t1_hbm, dstg_hbm, srcs_hbm, out1_hbm)

    return agg_kernel(table0, table1, srcg3, srcs3, dstg3, dsts3)


_DN = (((1,), (1,)), ((), ()))


def _emit_layer_pre(hb, w_ref, di_ref, hw_ref, hw2_ref, hcat_ref, cs_ref, i):
    hw = lax.dot_general(hb, w_ref[...], _DN,
                         preferred_element_type=jnp.float32)
    hw_ref[...] = hw
    dinv = lax.rsqrt(di_ref[...][:, 0:1] + 1.0)
    hw2_ref[...] = jnp.concatenate(
        [hw * dinv, jnp.zeros((BR, FC - F), jnp.float32)], axis=1)
    q = jnp.sum(hb * hb, axis=1, keepdims=True)
    hcat_ref[...] = jnp.concatenate(
        [hb, q, jnp.zeros((BR, FC - F - 1), jnp.float32)], axis=1)

    @pl.when(i == 0)
    def _():
        cs_ref[...] = jnp.sum(hb, axis=0, keepdims=True)

    @pl.when(i != 0)
    def _():
        cs_ref[...] += jnp.sum(hb, axis=0, keepdims=True)


def _combine(h_ref, hw_ref, a1_ref, a2_ref, xs_ref, di_ref, do_ref,
             cb_ref, cs_ref):
    hb = h_ref[...]
    gm = cs_ref[...] * (1.0 / N)
    dinv = lax.rsqrt(di_ref[...][:, 0:1] + 1.0)
    dout = do_ref[...][:, 0:1]
    a1 = a1_ref[...][:, :F]
    x_agg = jnp.maximum(
        dinv * a1 + (dinv * dinv) * hw_ref[...] + cb_ref[...], 0.0)
    a2full = a2_ref[...]
    agg2 = a2full[:, :F]
    s1 = a2full[:, F:F + 1]
    q = jnp.sum(hb * hb, axis=1, keepdims=True)
    dotv = jnp.sum(hb * agg2, axis=1, keepdims=True)
    gnum = dout * q + s1 - 2.0 * dotv
    gs = jnp.tanh(gnum / (dout + 1e-10))
    d = jnp.sum(jnp.abs(hb - gm) ** P, axis=1, keepdims=True)
    gq = 1.0 - jnp.tanh(d)
    return (hb + gs * x_agg + gq * xs_ref[...]) / (1.0 + gs + gq)


_ROWB = pl.BlockSpec((BR, F), lambda i: (i, 0))
_ROWC = pl.BlockSpec((BR, FC), lambda i: (i, 0))
_ROW16 = pl.BlockSpec((BR, 16), lambda i: (i, 0))
_WB = pl.BlockSpec((F, F), lambda i: (0, 0))
_B1 = pl.BlockSpec((1, F), lambda i: (0, 0))


def _tc_pre(x, enc_w, enc_b2, skip_w, conv_w, deg_in):
    def body(x_ref, ew_ref, eb_ref, sw_ref, w_ref, di_ref,
             h0_ref, xs_ref, hw_ref, hw2_ref, hcat_ref, cs_ref):
        i = pl.program_id(0)
        xb = x_ref[...]
        h0 = jnp.maximum(
            lax.dot_general(xb, ew_ref[...], _DN,
                            preferred_element_type=jnp.float32) + eb_ref[...],
            0.0)
        h0_ref[...] = h0
        xs_ref[...] = lax.dot_general(xb, sw_ref[...], _DN,
                                      preferred_element_type=jnp.float32)
        _emit_layer_pre(h0, w_ref, di_ref, hw_ref, hw2_ref, hcat_ref,
                        cs_ref, i)

    return pl.pallas_call(
        body,
        grid=(NB,),
        in_specs=[_ROWB, _WB, _B1, _WB, _WB, _ROW16],
        out_specs=[_ROWB, _ROWB, _ROWB, _ROWC, _ROWC, _B1],
        out_shape=[jax.ShapeDtypeStruct((N, F), jnp.float32),
                   jax.ShapeDtypeStruct((N, F), jnp.float32),
                   jax.ShapeDtypeStruct((N, F), jnp.float32),
                   jax.ShapeDtypeStruct((N, FC), jnp.float32),
                   jax.ShapeDtypeStruct((N, FC), jnp.float32),
                   jax.ShapeDtypeStruct((1, F), jnp.float32)],
    )(x, enc_w, enc_b2, skip_w, conv_w, deg_in)


def _tc_mid(h, hw, agg1, agg2cat, x_skip, deg_in, deg_out, cb, cs, conv_w):
    def body(h_ref, hw_ref, a1_ref, a2_ref, xs_ref, di_ref, do_ref,
             cb_ref, cs_ref, w_ref,
             h1_ref, hw1_ref, hw2_ref, hcat_ref, cs1_ref):
        i = pl.program_id(0)
        h1 = _combine(h_ref, hw_ref, a1_ref, a2_ref, xs_ref, di_ref, do_ref,
                      cb_ref, cs_ref)
        h1_ref[...] = h1
        _emit_layer_pre(h1, w_ref, di_ref, hw1_ref, hw2_ref, hcat_ref,
                        cs1_ref, i)

    return pl.pallas_call(
        body,
        grid=(NB,),
        in_specs=[_ROWB, _ROWB, _ROWC, _ROWC, _ROWB, _ROW16, _ROW16,
                  _B1, _B1, _WB],
        out_specs=[_ROWB, _ROWB, _ROWC, _ROWC, _B1],
        out_shape=[jax.ShapeDtypeStruct((N, F), jnp.float32),
                   jax.ShapeDtypeStruct((N, F), jnp.float32),
                   jax.ShapeDtypeStruct((N, FC), jnp.float32),
                   jax.ShapeDtypeStruct((N, FC), jnp.float32),
                   jax.ShapeDtypeStruct((1, F), jnp.float32)],
    )(h, hw, agg1, agg2cat, x_skip, deg_in, deg_out, cb, cs, conv_w)


def _tc_final(h, hw, agg1, agg2cat, x_skip, deg_in, deg_out, cb, cs,
              dec_w, dec_b2):
    def body(h_ref, hw_ref, a1_ref, a2_ref, xs_ref, di_ref, do_ref,
             cb_ref, cs_ref, dw_ref, db_ref, out_ref):
        h2 = _combine(h_ref, hw_ref, a1_ref, a2_ref, xs_ref, di_ref, do_ref,
                      cb_ref, cs_ref)
        out_ref[...] = lax.dot_general(
            h2, dw_ref[...], _DN,
            preferred_element_type=jnp.float32) + db_ref[...]

    return pl.pallas_call(
        body,
        grid=(NB,),
        in_specs=[_ROWB, _ROWB, _ROWC, _ROWC, _ROWB, _ROW16, _ROW16,
                  _B1, _B1,
                  pl.BlockSpec((NCLASS, F), lambda i: (0, 0)),
                  pl.BlockSpec((1, NCLASS), lambda i: (0, 0))],
        out_specs=pl.BlockSpec((BR, NCLASS), lambda i: (i, 0)),
        out_shape=jax.ShapeDtypeStruct((N, NCLASS), jnp.float32),
    )(h, hw, agg1, agg2cat, x_skip, deg_in, deg_out, cb, cs, dec_w, dec_b2)


def kernel(x, edge_index, enc_w, enc_b, skip_w, conv_w, conv_b, dec_w, dec_b):
    ept = E // NS
    src2 = edge_index[0].reshape(NS, ept)
    dst2 = edge_index[1].reshape(NS, ept)
    k = jnp.arange(NS * (EPT - ept), dtype=jnp.int32).reshape(NS, EPT - ept)
    padg = k % N
    pads = N + k % (NPAD - N)
    srcg3 = jnp.concatenate([src2, padg], 1).reshape(NS, CH, C)
    srcs3 = jnp.concatenate([src2, pads], 1).reshape(NS, CH, C)
    dstg3 = jnp.concatenate([dst2, padg], 1).reshape(NS, CH, C)
    dsts3 = jnp.concatenate([dst2, pads], 1).reshape(NS, CH, C)
    deg_in, deg_out = _sc_degrees(srcs3, dsts3)
    cb = conv_b.reshape(1, F)
    h, x_skip, hw, hw2, hcat, cs = _tc_pre(
        x, enc_w, enc_b.reshape(1, F), skip_w, conv_w, deg_in)
    agg1, agg2cat = _sc_aggregate(hw2, hcat, srcg3, srcs3, dstg3, dsts3)
    h, hw, hw2, hcat, cs = _tc_mid(
        h, hw, agg1, agg2cat, x_skip, deg_in, deg_out, cb, cs, conv_w)
    agg1, agg2cat = _sc_aggregate(hw2, hcat, srcg3, srcs3, dstg3, dsts3)
    return _tc_final(h, hw, agg1, agg2cat, x_skip, deg_in, deg_out, cb, cs,
                     dec_w, dec_b.reshape(1, NCLASS))


# speedup vs baseline: 1.1348x; 1.0263x over previous
import functools

import jax
import jax.numpy as jnp
from jax import lax
from jax.experimental import pallas as pl
from jax.experimental.pallas import tpu as pltpu
from jax.experimental.pallas import tpu_sc as plsc

N = 10000
NPAD = 10112
E = 320000
F = 128
FC = 144
NCLASS = 40
NS = 16
C = 32
NBUF = 8
G = 32
NG = 20
CH = NG * G
EPT = CH * C
RT = NPAD // NS
FLUSH = tuple((o, min(C, RT - o)) for o in range(0, RT, C))
BR = 1000
NB = N // BR
P = 2.5


def _zero_rows(ref, rows, width):
    def outer(i, _):
        for j in range(width // 16):
            ref[i, pl.ds(j * 16, 16)] = jnp.zeros((16,), jnp.float32)
        return 0
    lax.fori_loop(0, rows, outer, 0)


def _mesh():
    return plsc.VectorSubcoreMesh(core_axis_name="c", subcore_axis_name="s")


def _sc_degrees(srcs3, dsts3):
    @functools.partial(
        pl.kernel,
        out_type=(jax.ShapeDtypeStruct((NPAD, 16), jnp.float32),
                  jax.ShapeDtypeStruct((NPAD, 16), jnp.float32)),
        mesh=_mesh(),
        scratch_types=[
            pltpu.VMEM((CH, C), jnp.int32),
            pltpu.VMEM((C, 16), jnp.float32),
            pltpu.VMEM((128, 16), jnp.float32),
            pltpu.VMEM_SHARED((NPAD, 16), jnp.float32),
        ],
        compiler_params=pltpu.CompilerParams(use_tc_tiling_on_sc=False),
    )
    def deg_kernel(src_hbm, dst_hbm, degin_hbm, degout_hbm,
                   idx_v, ones_v, stage_v, acc_sh):
        c = lax.axis_index("c")
        s = lax.axis_index("s")

        def fill_ones(i, _):
            ones_v[i, :] = jnp.ones((16,), jnp.float32)
            return 0
        lax.fori_loop(0, C, fill_ones, 0)
        _zero_rows(stage_v, 128, 16)

        base = s * RT
        for off, sz in FLUSH:
            pltpu.sync_copy(stage_v.at[:sz], acc_sh.at[pl.ds(base + off, sz)])
        plsc.subcore_barrier()

        def run(idx_hbm, out_hbm):
            pltpu.sync_copy(idx_hbm.at[s], idx_v)

            def chunk(j, _):
                pltpu.sync_copy(ones_v, acc_sh.at[idx_v.at[j]], add=True)
                return 0
            lax.fori_loop(0, CH, chunk, 0)
            plsc.subcore_barrier()
            for off, sz in FLUSH:
                r0 = base + off
                pltpu.sync_copy(acc_sh.at[pl.ds(r0, sz)], stage_v.at[:sz])
                pltpu.sync_copy(stage_v.at[:sz], out_hbm.at[pl.ds(r0, sz)])

        @pl.when(c == 0)
        def _():
            run(dst_hbm, degin_hbm)

        @pl.when(c == 1)
        def _():
            run(src_hbm, degout_hbm)

    return deg_kernel(srcs3, dsts3)


def _sc_aggregate(table0, table1, srcg3, srcs3, dstg3, dsts3):
    @functools.partial(
        pl.kernel,
        out_type=(jax.ShapeDtypeStruct((NPAD, FC), jnp.float32),
                  jax.ShapeDtypeStruct((NPAD, FC), jnp.float32)),
        mesh=_mesh(),
        scratch_types=[
            pltpu.VMEM((G, C), jnp.int32),
            pltpu.VMEM((G, C), jnp.int32),
            pltpu.VMEM((NBUF, C, FC), jnp.float32),
            pltpu.VMEM_SHARED((NPAD, FC), jnp.float32),
            pltpu.SemaphoreType.DMA((NBUF,)),
            pltpu.SemaphoreType.DMA((NBUF,)),
        ],
        compiler_params=pltpu.CompilerParams(use_tc_tiling_on_sc=False),
    )
    def agg_kernel(t0_hbm, t1_hbm, srcg_hbm, srcs_hbm, dstg_hbm, dsts_hbm,
                   out0_hbm, out1_hbm,
                   idxg_v, idxs_v, bufn_v, acc_sh,
                   gsem, ssem):
        c = lax.axis_index("c")
        s = lax.axis_index("s")
        bufs = tuple(bufn_v.at[k] for k in range(NBUF))
        buf0_v = bufn_v.at[0]

        _zero_rows(buf0_v, C, FC)
        base = s * RT
        for off, sz in FLUSH:
            pltpu.sync_copy(buf0_v.at[:sz], acc_sh.at[pl.ds(base + off, sz)])
        plsc.subcore_barrier()

        def run(table_hbm, ig_hbm, is_hbm, out_hbm):
            def group(g, _):
                pltpu.sync_copy(ig_hbm.at[s, pl.ds(g * G, G)], idxg_v)
                pltpu.sync_copy(is_hbm.at[s, pl.ds(g * G, G)], idxs_v)
                gd = [None] * G
                sd = [None] * G
                for k in range(NBUF - 1):
                    gd[k] = pltpu.async_copy(
                        table_hbm.at[idxg_v.at[k]], bufs[k], gsem.at[k])
                for j in range(G):
                    b = j % NBUF
                    gd[j].wait()
                    sd[j] = pltpu.async_copy(
                        bufs[b], acc_sh.at[idxs_v.at[j]], ssem.at[b],
                        add=True)
                    jn = j + NBUF - 1
                    if jn < G:
                        if j >= 1:
                            sd[j - 1].wait()
                        gd[jn] = pltpu.async_copy(
                            table_hbm.at[idxg_v.at[jn]], bufs[jn % NBUF],
                            gsem.at[jn % NBUF])
                for j in range(G - NBUF, G):
                    if j >= 0:
                        sd[j].wait()
                return 0
            lax.fori_loop(0, NG, group, 0)
            plsc.subcore_barrier()
            for off, sz in FLUSH:
                r0 = base + off
                pltpu.sync_copy(acc_sh.at[pl.ds(r0, sz)], buf0_v.at[:sz])
                pltpu.sync_copy(buf0_v.at[:sz], out_hbm.at[pl.ds(r0, sz)])

        @pl.when(c == 0)
        def _():
            run(t0_hbm, srcg_hbm, dsts_hbm, out0_hbm)

        @pl.when(c == 1)
        def _():
            run(t1_hbm, dstg_hbm, srcs_hbm, out1_hbm)

    return agg_kernel(table0, table1, srcg3, srcs3, dstg3, dsts3)


_DN = (((1,), (1,)), ((), ()))


def _emit_layer_pre(hb, w_ref, di_ref, hw_ref, hw2_ref, hcat_ref, cs_ref, i):
    hw = lax.dot_general(hb, w_ref[...], _DN,
                         preferred_element_type=jnp.float32)
    hw_ref[...] = hw
    dinv = lax.rsqrt(di_ref[...][:, 0:1] + 1.0)
    hw2_ref[...] = jnp.concatenate(
        [hw * dinv, jnp.zeros((BR, FC - F), jnp.float32)], axis=1)
    q = jnp.sum(hb * hb, axis=1, keepdims=True)
    hcat_ref[...] = jnp.concatenate(
        [hb, q, jnp.zeros((BR, FC - F - 1), jnp.float32)], axis=1)

    @pl.when(i == 0)
    def _():
        cs_ref[...] = jnp.sum(hb, axis=0, keepdims=True)

    @pl.when(i != 0)
    def _():
        cs_ref[...] += jnp.sum(hb, axis=0, keepdims=True)


def _combine(h_ref, hw_ref, a1_ref, a2_ref, xs_ref, di_ref, do_ref,
             cb_ref, cs_ref):
    hb = h_ref[...]
    gm = cs_ref[...] * (1.0 / N)
    dinv = lax.rsqrt(di_ref[...][:, 0:1] + 1.0)
    dout = do_ref[...][:, 0:1]
    a1 = a1_ref[...][:, :F]
    x_agg = jnp.maximum(
        dinv * a1 + (dinv * dinv) * hw_ref[...] + cb_ref[...], 0.0)
    a2full = a2_ref[...]
    agg2 = a2full[:, :F]
    s1 = a2full[:, F:F + 1]
    q = jnp.sum(hb * hb, axis=1, keepdims=True)
    dotv = jnp.sum(hb * agg2, axis=1, keepdims=True)
    gnum = dout * q + s1 - 2.0 * dotv
    gs = jnp.tanh(gnum / (dout + 1e-10))
    d = jnp.sum(jnp.abs(hb - gm) ** P, axis=1, keepdims=True)
    gq = 1.0 - jnp.tanh(d)
    return (hb + gs * x_agg + gq * xs_ref[...]) / (1.0 + gs + gq)


_ROWB = pl.BlockSpec((BR, F), lambda i: (i, 0))
_ROWC = pl.BlockSpec((BR, FC), lambda i: (i, 0))
_ROW16 = pl.BlockSpec((BR, 16), lambda i: (i, 0))
_WB = pl.BlockSpec((F, F), lambda i: (0, 0))
_B1 = pl.BlockSpec((1, F), lambda i: (0, 0))


def _tc_pre(x, enc_w, enc_b2, skip_w, conv_w):
    def body(x_ref, ew_ref, eb_ref, sw_ref, w_ref,
             h0_ref, xs_ref, hw_ref, hcat_ref, cs_ref):
        i = pl.program_id(0)
        xb = x_ref[...]
        h0 = jnp.maximum(
            lax.dot_general(xb, ew_ref[...], _DN,
                            preferred_element_type=jnp.float32) + eb_ref[...],
            0.0)
        h0_ref[...] = h0
        xs_ref[...] = lax.dot_general(xb, sw_ref[...], _DN,
                                      preferred_element_type=jnp.float32)
        hw_ref[...] = lax.dot_general(h0, w_ref[...], _DN,
                                      preferred_element_type=jnp.float32)
        q = jnp.sum(h0 * h0, axis=1, keepdims=True)
        hcat_ref[...] = jnp.concatenate(
            [h0, q, jnp.zeros((BR, FC - F - 1), jnp.float32)], axis=1)

        @pl.when(i == 0)
        def _():
            cs_ref[...] = jnp.sum(h0, axis=0, keepdims=True)

        @pl.when(i != 0)
        def _():
            cs_ref[...] += jnp.sum(h0, axis=0, keepdims=True)

    return pl.pallas_call(
        body,
        grid=(NB,),
        in_specs=[_ROWB, _WB, _B1, _WB, _WB],
        out_specs=[_ROWB, _ROWB, _ROWB, _ROWC, _B1],
        out_shape=[jax.ShapeDtypeStruct((N, F), jnp.float32),
                   jax.ShapeDtypeStruct((N, F), jnp.float32),
                   jax.ShapeDtypeStruct((N, F), jnp.float32),
                   jax.ShapeDtypeStruct((N, FC), jnp.float32),
                   jax.ShapeDtypeStruct((1, F), jnp.float32)],
    )(x, enc_w, enc_b2, skip_w, conv_w)


def _tc_scale(hw, deg_in):
    def body(hw_ref, di_ref, hw2_ref):
        dinv = lax.rsqrt(di_ref[...][:, 0:1] + 1.0)
        hw2_ref[...] = jnp.concatenate(
            [hw_ref[...] * dinv, jnp.zeros((BR, FC - F), jnp.float32)],
            axis=1)

    return pl.pallas_call(
        body,
        grid=(NB,),
        in_specs=[_ROWB, _ROW16],
        out_specs=_ROWC,
        out_shape=jax.ShapeDtypeStruct((N, FC), jnp.float32),
    )(hw, deg_in)


def _tc_mid(h, hw, agg1, agg2cat, x_skip, deg_in, deg_out, cb, cs, conv_w):
    def body(h_ref, hw_ref, a1_ref, a2_ref, xs_ref, di_ref, do_ref,
             cb_ref, cs_ref, w_ref,
             h1_ref, hw1_ref, hw2_ref, hcat_ref, cs1_ref):
        i = pl.program_id(0)
        h1 = _combine(h_ref, hw_ref, a1_ref, a2_ref, xs_ref, di_ref, do_ref,
                      cb_ref, cs_ref)
        h1_ref[...] = h1
        _emit_layer_pre(h1, w_ref, di_ref, hw1_ref, hw2_ref, hcat_ref,
                        cs1_ref, i)

    return pl.pallas_call(
        body,
        grid=(NB,),
        in_specs=[_ROWB, _ROWB, _ROWC, _ROWC, _ROWB, _ROW16, _ROW16,
                  _B1, _B1, _WB],
        out_specs=[_ROWB, _ROWB, _ROWC, _ROWC, _B1],
        out_shape=[jax.ShapeDtypeStruct((N, F), jnp.float32),
                   jax.ShapeDtypeStruct((N, F), jnp.float32),
                   jax.ShapeDtypeStruct((N, FC), jnp.float32),
                   jax.ShapeDtypeStruct((N, FC), jnp.float32),
                   jax.ShapeDtypeStruct((1, F), jnp.float32)],
    )(h, hw, agg1, agg2cat, x_skip, deg_in, deg_out, cb, cs, conv_w)


def _tc_final(h, hw, agg1, agg2cat, x_skip, deg_in, deg_out, cb, cs,
              dec_w, dec_b2):
    def body(h_ref, hw_ref, a1_ref, a2_ref, xs_ref, di_ref, do_ref,
             cb_ref, cs_ref, dw_ref, db_ref, out_ref):
        h2 = _combine(h_ref, hw_ref, a1_ref, a2_ref, xs_ref, di_ref, do_ref,
                      cb_ref, cs_ref)
        out_ref[...] = lax.dot_general(
            h2, dw_ref[...], _DN,
            preferred_element_type=jnp.float32) + db_ref[...]

    return pl.pallas_call(
        body,
        grid=(NB,),
        in_specs=[_ROWB, _ROWB, _ROWC, _ROWC, _ROWB, _ROW16, _ROW16,
                  _B1, _B1,
                  pl.BlockSpec((NCLASS, F), lambda i: (0, 0)),
                  pl.BlockSpec((1, NCLASS), lambda i: (0, 0))],
        out_specs=pl.BlockSpec((BR, NCLASS), lambda i: (i, 0)),
        out_shape=jax.ShapeDtypeStruct((N, NCLASS), jnp.float32),
    )(h, hw, agg1, agg2cat, x_skip, deg_in, deg_out, cb, cs, dec_w, dec_b2)


def kernel(x, edge_index, enc_w, enc_b, skip_w, conv_w, conv_b, dec_w, dec_b):
    ept = E // NS
    src2 = edge_index[0].reshape(NS, ept)
    dst2 = edge_index[1].reshape(NS, ept)
    k = jnp.arange(NS * (EPT - ept), dtype=jnp.int32).reshape(NS, EPT - ept)
    padg = k % N
    pads = N + k % (NPAD - N)
    srcg3 = jnp.concatenate([src2, padg], 1).reshape(NS, CH, C)
    srcs3 = jnp.concatenate([src2, pads], 1).reshape(NS, CH, C)
    dstg3 = jnp.concatenate([dst2, padg], 1).reshape(NS, CH, C)
    dsts3 = jnp.concatenate([dst2, pads], 1).reshape(NS, CH, C)
    deg_in, deg_out = _sc_degrees(srcs3, dsts3)
    cb = conv_b.reshape(1, F)
    h, x_skip, hw, hcat, cs = _tc_pre(
        x, enc_w, enc_b.reshape(1, F), skip_w, conv_w)
    hw2 = _tc_scale(hw, deg_in)
    agg1, agg2cat = _sc_aggregate(hw2, hcat, srcg3, srcs3, dstg3, dsts3)
    h, hw, hw2, hcat, cs = _tc_mid(
        h, hw, agg1, agg2cat, x_skip, deg_in, deg_out, cb, cs, conv_w)
    agg1, agg2cat = _sc_aggregate(hw2, hcat, srcg3, srcs3, dstg3, dsts3)
    return _tc_final(h, hw, agg1, agg2cat, x_skip, deg_in, deg_out, cb, cs,
                     dec_w, dec_b.reshape(1, NCLASS))
